# Initial kernel scaffold; baseline (speedup 1.0000x reference)
import jax
import jax.numpy as jnp
from jax.experimental import pallas as pl


def kernel(hidden_states, router_W, fc1_W, fc2_W):
    raise NotImplementedError("write your pallas kernel here")


# speedup vs baseline: 1.7149x; 1.7149x over previous
import functools

import jax
import jax.numpy as jnp
from jax import lax
from jax.experimental import pallas as pl
from jax.experimental.pallas import tpu as pltpu
from jax.experimental.pallas import tpu_sc as plsc

H = 1024
FF = 4096
E = 8
K = 2
BT = 256
FT = 512


def _router_body(x_ref, rw_ref, i1_ref, i2_ref, w1_ref, w2_ref):
    x = x_ref[...]
    rw = rw_ref[...]
    logits = lax.dot_general(x, rw, (((1,), (1,)), ((), ())),
                             preferred_element_type=jnp.float32)
    m = jnp.max(logits, axis=1, keepdims=True)
    p = jnp.exp(logits - m)
    probs = p / jnp.sum(p, axis=1, keepdims=True)
    iota = lax.broadcasted_iota(jnp.int32, probs.shape, 1)
    v1 = jnp.max(probs, axis=1, keepdims=True)
    i1 = jnp.min(jnp.where(probs == v1, iota, E), axis=1, keepdims=True)
    masked = jnp.where(iota == i1, -1.0, probs)
    v2 = jnp.max(masked, axis=1, keepdims=True)
    i2 = jnp.min(jnp.where(masked == v2, iota, E), axis=1, keepdims=True)
    denom = jnp.clip(v1 + v2, 1e-9, None)
    i1_ref[...] = i1
    i2_ref[...] = i2
    w1_ref[...] = v1 / denom
    w2_ref[...] = v2 / denom


def _router(x, router_W, interpret=False):
    s = x.shape[0]
    return pl.pallas_call(
        _router_body,
        out_shape=(
            jax.ShapeDtypeStruct((s, 1), jnp.int32),
            jax.ShapeDtypeStruct((s, 1), jnp.int32),
            jax.ShapeDtypeStruct((s, 1), jnp.float32),
            jax.ShapeDtypeStruct((s, 1), jnp.float32),
        ),
        interpret=interpret,
    )(x, router_W)


def _dispatch_meta(i1, i2, w1, w2, num_blocks, p_total):
    s = i1.shape[0]
    a = s * K
    e_flat = jnp.stack([i1[:, 0], i2[:, 0]], axis=1).reshape(-1)
    w_flat = jnp.stack([w1[:, 0], w2[:, 0]], axis=1).reshape(-1)
    tok_of_a = (jnp.arange(a, dtype=jnp.int32) // K).astype(jnp.int32)
    onehot = (e_flat[:, None] == jnp.arange(E, dtype=jnp.int32)[None, :])
    counts = jnp.sum(onehot, axis=0).astype(jnp.int32)
    g_start = jnp.concatenate(
        [jnp.zeros((1,), jnp.int32), jnp.cumsum(counts)[:-1].astype(jnp.int32)])
    padded = ((counts + BT - 1) // BT) * BT
    p_start = jnp.concatenate(
        [jnp.zeros((1,), jnp.int32), jnp.cumsum(padded)[:-1].astype(jnp.int32)])
    order = jnp.argsort(e_flat)
    e_sorted = e_flat[order]
    r = jnp.arange(a, dtype=jnp.int32)
    dst = p_start[e_sorted] + (r - g_start[e_sorted])
    tok_padded = jnp.zeros((p_total,), jnp.int32).at[dst].set(tok_of_a[order])
    w_padded = jnp.zeros((p_total,), jnp.float32).at[dst].set(w_flat[order])
    pos_of_a = jnp.zeros((a,), jnp.int32).at[order].set(dst)
    pos_a = pos_of_a[0::K]
    pos_b = pos_of_a[1::K]
    bb = jnp.arange(num_blocks, dtype=jnp.int32)[:, None]
    sb = (p_start // BT)[None, :]
    eb = ((p_start + padded) // BT)[None, :]
    in_grp = (bb >= sb) & (bb < eb)
    block_expert = jnp.sum(
        jnp.where(in_grp, jnp.arange(E, dtype=jnp.int32)[None, :], 0), axis=1)
    return tok_padded, w_padded, pos_a, pos_b, block_expert.astype(jnp.int32)


def _ffn_body(be_ref, x_ref, w_ref, fc1_ref, fc2_ref, o_ref, acc_ref):
    j = pl.program_id(1)
    nj = pl.num_programs(1)

    @pl.when(j == 0)
    def _():
        acc_ref[...] = jnp.zeros_like(acc_ref)

    x = x_ref[...]
    h = lax.dot_general(x, fc1_ref[0], (((1,), (1,)), ((), ())),
                        preferred_element_type=jnp.float32)
    g = 0.5 * h * (1.0 + lax.erf(h * 0.7071067811865476))
    acc_ref[...] += lax.dot_general(g, fc2_ref[0], (((1,), (1,)), ((), ())),
                                    preferred_element_type=jnp.float32)

    @pl.when(j == nj - 1)
    def _():
        o_ref[...] = acc_ref[...] * w_ref[...]


def _grouped_ffn(block_expert, x_sorted, w_padded, fc1_W, fc2_W,
                 interpret=False):
    p_total = x_sorted.shape[0]
    nb = p_total // BT
    nj = FF // FT
    grid_spec = pltpu.PrefetchScalarGridSpec(
        num_scalar_prefetch=1,
        grid=(nb, nj),
        in_specs=[
            pl.BlockSpec((BT, H), lambda b, j, be: (b, 0)),
            pl.BlockSpec((BT, 1), lambda b, j, be: (b, 0)),
            pl.BlockSpec((1, FT, H), lambda b, j, be: (be[b], j, 0)),
            pl.BlockSpec((1, H, FT), lambda b, j, be: (be[b], 0, j)),
        ],
        out_specs=pl.BlockSpec((BT, H), lambda b, j, be: (b, 0)),
        scratch_shapes=[pltpu.VMEM((BT, H), jnp.float32)],
    )
    return pl.pallas_call(
        _ffn_body,
        grid_spec=grid_spec,
        out_shape=jax.ShapeDtypeStruct((p_total, H), jnp.float32),
        compiler_params=pltpu.CompilerParams(
            dimension_semantics=("arbitrary", "arbitrary")),
        interpret=interpret,
    )(block_expert, x_sorted, w_padded, fc1_W, fc2_W)


def _sc_gather(x, tok_padded, p_total):
    info = plsc.get_sparse_core_info()
    nc, ns = info.num_cores, info.num_subcores
    nw = nc * ns
    per_w = p_total // nw
    ch = per_w
    while ch * H * 4 > 256 * 1024 or per_w % ch:
        ch -= 1
    mesh = plsc.VectorSubcoreMesh(core_axis_name="c", subcore_axis_name="s")

    @functools.partial(
        pl.kernel, mesh=mesh,
        out_type=jax.ShapeDtypeStruct((p_total, H), jnp.float32),
        scratch_types=[
            pltpu.VMEM((per_w,), jnp.int32),
            pltpu.VMEM((ch, H), jnp.float32),
            pltpu.SemaphoreType.DMA,
        ])
    def gk(tab_hbm, idx_hbm, out_hbm, idx_v, rows_v, sem):
        wid = lax.axis_index("s") * nc + lax.axis_index("c")
        base = wid * per_w
        pltpu.sync_copy(idx_hbm.at[pl.ds(base, per_w)], idx_v)
        for c in range(per_w // ch):
            pltpu.async_copy(
                tab_hbm.at[idx_v.at[pl.ds(c * ch, ch)]], rows_v, sem).wait()
            pltpu.sync_copy(rows_v, out_hbm.at[pl.ds(base + c * ch, ch)])

    return gk(x, tok_padded)


def _sc_combine(ys, pos_a, pos_b):
    s = pos_a.shape[0]
    info = plsc.get_sparse_core_info()
    nc, ns = info.num_cores, info.num_subcores
    nw = nc * ns
    per_w = s // nw
    ch = per_w
    while ch * H * 4 > 128 * 1024 or per_w % ch:
        ch -= 1
    mesh = plsc.VectorSubcoreMesh(core_axis_name="c", subcore_axis_name="s")

    @functools.partial(
        pl.kernel, mesh=mesh,
        out_type=jax.ShapeDtypeStruct((s, H), jnp.float32),
        scratch_types=[
            pltpu.VMEM((per_w,), jnp.int32),
            pltpu.VMEM((per_w,), jnp.int32),
            pltpu.VMEM((ch, H), jnp.float32),
            pltpu.VMEM((ch, H), jnp.float32),
            pltpu.SemaphoreType.DMA,
        ])
    def ck(ys_hbm, pa_hbm, pb_hbm, out_hbm, ia_v, ib_v, ra_v, rb_v, sem):
        wid = lax.axis_index("s") * nc + lax.axis_index("c")
        base = wid * per_w
        pltpu.sync_copy(pa_hbm.at[pl.ds(base, per_w)], ia_v)
        pltpu.sync_copy(pb_hbm.at[pl.ds(base, per_w)], ib_v)
        for c in range(per_w // ch):
            pltpu.async_copy(
                ys_hbm.at[ia_v.at[pl.ds(c * ch, ch)]], ra_v, sem).wait()
            pltpu.async_copy(
                ys_hbm.at[ib_v.at[pl.ds(c * ch, ch)]], rb_v, sem).wait()

            def row(rr, carry):
                def lane(kk, carry2):
                    sl = pl.ds(kk * 16, 16)
                    ra_v[rr, sl] = ra_v[rr, sl] + rb_v[rr, sl]
                    return carry2
                return lax.fori_loop(0, H // 16, lane, carry)

            lax.fori_loop(0, ch, row, 0)
            pltpu.sync_copy(ra_v, out_hbm.at[pl.ds(base + c * ch, ch)])

    return ck(ys, pos_a, pos_b)


def kernel(hidden_states, router_W, fc1_W, fc2_W):
    orig_shape = hidden_states.shape
    x = hidden_states.reshape(-1, H)
    s = x.shape[0]
    p_total = s * K + E * BT
    nb = p_total // BT
    i1, i2, w1, w2 = _router(x, router_W)
    tok_padded, w_padded, pos_a, pos_b, block_expert = _dispatch_meta(
        i1, i2, w1, w2, nb, p_total)
    x_sorted = _sc_gather(x, tok_padded, p_total)
    ys = _grouped_ffn(block_expert, x_sorted, w_padded.reshape(p_total, 1),
                      fc1_W, fc2_W)
    y = _sc_combine(ys, pos_a, pos_b)
    return y.reshape(orig_shape)
